# baseline (speedup 1.0000x reference)
import functools

import jax
import jax.numpy as jnp
from jax import lax
from jax.experimental import pallas as pl
from jax.experimental.pallas import tpu as pltpu
from jax.experimental.pallas import tpu_sc as plsc

HID = 768
TOK = 4 * 2048

NC = 2
NS = 16
NW = NC * NS
TOK_PER_W = TOK // NW
CHUNK = 32
NCHUNK = TOK_PER_W // CHUNK
LANES = 16
VPT = HID // LANES


def _sc_body(ids_hbm, pids_hbm, tids_hbm, wtab, ptab, ttab, out_hbm,
             idx_w, idx_p, idx_t, rows_w, rows_p, rows_t, sem):
    wid = lax.axis_index("s") * NC + lax.axis_index("c")
    base = wid * TOK_PER_W
    for c in range(NCHUNK):
        tok0 = base + c * CHUNK
        pltpu.sync_copy(ids_hbm.at[pl.ds(tok0, CHUNK)], idx_w)
        pltpu.sync_copy(pids_hbm.at[pl.ds(tok0, CHUNK)], idx_p)
        pltpu.sync_copy(tids_hbm.at[pl.ds(tok0, CHUNK)], idx_t)
        cw = pltpu.async_copy(wtab.at[idx_w], rows_w, sem)
        cp = pltpu.async_copy(ptab.at[idx_p], rows_p, sem)
        ct = pltpu.async_copy(ttab.at[idx_t], rows_t, sem)
        cw.wait()
        cp.wait()
        ct.wait()

        def add_rows(j, carry):
            for k in range(VPT):
                sl = pl.ds(k * LANES, LANES)
                rows_w[j, sl] = rows_w[j, sl] + rows_p[j, sl] + rows_t[j, sl]
            return carry

        lax.fori_loop(0, CHUNK, add_rows, 0)
        pltpu.sync_copy(rows_w, out_hbm.at[pl.ds(tok0, CHUNK)])


def kernel(input_ids, position_ids, token_type_ids, word_embeddings,
           token_type_embeddings, position_embeddings):
    b, s = input_ids.shape
    ids = input_ids.reshape(-1).astype(jnp.int32)
    pids = position_ids.reshape(-1).astype(jnp.int32)
    tids = token_type_ids.reshape(-1).astype(jnp.int32)

    mesh = plsc.VectorSubcoreMesh(core_axis_name="c", subcore_axis_name="s")
    run = functools.partial(
        pl.kernel,
        mesh=mesh,
        out_type=jax.ShapeDtypeStruct((TOK, HID), jnp.float32),
        scratch_types=[
            pltpu.VMEM((CHUNK,), jnp.int32),
            pltpu.VMEM((CHUNK,), jnp.int32),
            pltpu.VMEM((CHUNK,), jnp.int32),
            pltpu.VMEM((CHUNK, HID), jnp.float32),
            pltpu.VMEM((CHUNK, HID), jnp.float32),
            pltpu.VMEM((CHUNK, HID), jnp.float32),
            pltpu.SemaphoreType.DMA,
        ],
    )(_sc_body)

    out = run(ids, pids, tids,
              word_embeddings.astype(jnp.float32),
              position_embeddings.astype(jnp.float32),
              token_type_embeddings.astype(jnp.float32))
    return out.reshape(b, s, HID)


# speedup vs baseline: 2.5938x; 2.5938x over previous
import functools

import jax
import jax.numpy as jnp
from jax import lax
from jax.experimental import pallas as pl
from jax.experimental.pallas import tpu as pltpu
from jax.experimental.pallas import tpu_sc as plsc

HID = 768
TOK = 4 * 2048

NC = 2
NS = 16
NW = NC * NS
TOK_PER_W = TOK // NW
CHUNK = 32
NCHUNK = TOK_PER_W // CHUNK
LANES = 16
VPT = HID // LANES


def _sc_body(ids_hbm, pids_hbm, tids_hbm, wtab, ptab, ttab, out_hbm,
             idx_w, idx_p, idx_t, type_v,
             bufw0, bufw1, bufp0, bufp1,
             semw0, semw1, semp0, semp1, semo0, semo1):
    wid = lax.axis_index("s") * NC + lax.axis_index("c")
    row0 = wid * NCHUNK
    base = wid * TOK_PER_W

    bufw = (bufw0, bufw1)
    bufp = (bufp0, bufp1)
    semw = (semw0, semw1)
    semp = (semp0, semp1)
    semo = (semo0, semo1)

    pltpu.sync_copy(ids_hbm.at[pl.ds(row0, NCHUNK)], idx_w)
    pltpu.sync_copy(pids_hbm.at[pl.ds(row0, NCHUNK)], idx_p)
    pltpu.sync_copy(tids_hbm.at[pl.ds(base, TOK_PER_W)], idx_t)
    pltpu.sync_copy(ttab, type_v)

    def start_gathers(c):
        b = c % 2
        cw = pltpu.async_copy(wtab.at[idx_w.at[c]], bufw[b], semw[b])
        cp = pltpu.async_copy(ptab.at[idx_p.at[c]], bufp[b], semp[b])
        return cw, cp

    kvecs = [lax.iota(jnp.int32, LANES) + k * LANES for k in range(VPT)]

    def compute(c, b):
        def add_rows(j, carry):
            jv = jnp.full((LANES,), c * CHUNK + j, jnp.int32)
            tsp = plsc.load_gather(idx_t, [jv])
            tbase = tsp * HID
            for k in range(VPT):
                sl = pl.ds(k * LANES, LANES)
                trow = plsc.load_gather(type_v, [tbase + kvecs[k]])
                bufw[b][j, sl] = bufw[b][j, sl] + bufp[b][j, sl] + trow
            return carry

        lax.fori_loop(0, CHUNK, add_rows, 0)

    gath = {0: start_gathers(0)}
    outc = [None, None]
    for c in range(NCHUNK):
        b = c % 2
        if c + 1 < NCHUNK:
            if outc[(c + 1) % 2] is not None:
                outc[(c + 1) % 2].wait()
            gath[c + 1] = start_gathers(c + 1)
        cw, cp = gath.pop(c)
        cw.wait()
        cp.wait()
        compute(c, b)
        outc[b] = pltpu.async_copy(
            bufw[b], out_hbm.at[pl.ds(base + c * CHUNK, CHUNK)], semo[b])
    outc[0].wait()
    outc[1].wait()


def kernel(input_ids, position_ids, token_type_ids, word_embeddings,
           token_type_embeddings, position_embeddings):
    b, s = input_ids.shape
    ids = input_ids.reshape(TOK // CHUNK, CHUNK).astype(jnp.int32)
    pids = position_ids.reshape(TOK // CHUNK, CHUNK).astype(jnp.int32)
    tids = token_type_ids.reshape(TOK).astype(jnp.int32)

    mesh = plsc.VectorSubcoreMesh(core_axis_name="c", subcore_axis_name="s")
    run = functools.partial(
        pl.kernel,
        mesh=mesh,
        out_type=jax.ShapeDtypeStruct((TOK, HID), jnp.float32),
        compiler_params=pltpu.CompilerParams(needs_layout_passes=False),
        scratch_types=[
            pltpu.VMEM((NCHUNK, CHUNK), jnp.int32),
            pltpu.VMEM((NCHUNK, CHUNK), jnp.int32),
            pltpu.VMEM((TOK_PER_W,), jnp.int32),
            pltpu.VMEM((2 * HID,), jnp.float32),
            pltpu.VMEM((CHUNK, HID), jnp.float32),
            pltpu.VMEM((CHUNK, HID), jnp.float32),
            pltpu.VMEM((CHUNK, HID), jnp.float32),
            pltpu.VMEM((CHUNK, HID), jnp.float32),
            pltpu.SemaphoreType.DMA,
            pltpu.SemaphoreType.DMA,
            pltpu.SemaphoreType.DMA,
            pltpu.SemaphoreType.DMA,
            pltpu.SemaphoreType.DMA,
            pltpu.SemaphoreType.DMA,
        ],
    )(_sc_body)

    out = run(ids, pids, tids,
              word_embeddings.astype(jnp.float32),
              position_embeddings.astype(jnp.float32),
              token_type_embeddings.astype(jnp.float32).reshape(-1))
    return out.reshape(b, s, HID)


# speedup vs baseline: 4.1872x; 1.6143x over previous
import functools

import jax
import jax.numpy as jnp
from jax import lax
from jax.experimental import pallas as pl
from jax.experimental.pallas import tpu as pltpu
from jax.experimental.pallas import tpu_sc as plsc

HID = 768
TOK = 4 * 2048

NC = 2
NS = 16
NW = NC * NS
TOK_PER_W = TOK // NW
CHUNK = 32
NCHUNK = TOK_PER_W // CHUNK
LANES = 16
VPT = HID // LANES


def _sc_body(ids_hbm, pids_hbm, tidsf_hbm, wtab, ptab, ttab, out_hbm,
             idx_w, idx_p, tf_vmem, type_v,
             bufw0, bufw1, bufp0, bufp1,
             semw0, semw1, semp0, semp1, semo0, semo1):
    wid = lax.axis_index("s") * NC + lax.axis_index("c")
    row0 = wid * NCHUNK
    base = wid * TOK_PER_W

    bufw = (bufw0, bufw1)
    bufp = (bufp0, bufp1)
    semw = (semw0, semw1)
    semp = (semp0, semp1)
    semo = (semo0, semo1)

    pltpu.sync_copy(ids_hbm.at[pl.ds(row0, NCHUNK)], idx_w)
    pltpu.sync_copy(pids_hbm.at[pl.ds(row0, NCHUNK)], idx_p)
    pltpu.sync_copy(tidsf_hbm.at[pl.ds(base, TOK_PER_W)], tf_vmem)
    pltpu.sync_copy(ttab, type_v)

    def start_gathers(c):
        b = c % 2
        cw = pltpu.async_copy(wtab.at[idx_w.at[c]], bufw[b], semw[b])
        cp = pltpu.async_copy(ptab.at[idx_p.at[c]], bufp[b], semp[b])
        return cw, cp

    def compute(c, b):
        def add_k(k, carry):
            sl = pl.ds(k * LANES, LANES)
            t0k = type_v[0, sl]
            dk = type_v[1, sl] - t0k
            for g in range(CHUNK // LANES):
                tfv = tf_vmem[pl.ds(c * CHUNK + g * LANES, LANES)]
                for l in range(LANES):
                    j = g * LANES + l
                    tf = jnp.full((LANES,), tfv[l], jnp.float32)
                    bufw[b][j, sl] = (bufw[b][j, sl] + bufp[b][j, sl]
                                      + (t0k + tf * dk))
            return carry

        lax.fori_loop(0, VPT, add_k, 0)

    gath = {0: start_gathers(0)}
    outc = [None, None]
    for c in range(NCHUNK):
        b = c % 2
        if c + 1 < NCHUNK:
            if outc[(c + 1) % 2] is not None:
                outc[(c + 1) % 2].wait()
            gath[c + 1] = start_gathers(c + 1)
        cw, cp = gath.pop(c)
        cw.wait()
        cp.wait()
        compute(c, b)
        outc[b] = pltpu.async_copy(
            bufw[b], out_hbm.at[pl.ds(base + c * CHUNK, CHUNK)], semo[b])
    outc[0].wait()
    outc[1].wait()


def kernel(input_ids, position_ids, token_type_ids, word_embeddings,
           token_type_embeddings, position_embeddings):
    b, s = input_ids.shape
    ids = input_ids.reshape(TOK // CHUNK, CHUNK).astype(jnp.int32)
    pids = position_ids.reshape(TOK // CHUNK, CHUNK).astype(jnp.int32)
    tidsf = token_type_ids.reshape(TOK).astype(jnp.float32)

    mesh = plsc.VectorSubcoreMesh(core_axis_name="c", subcore_axis_name="s")
    run = functools.partial(
        pl.kernel,
        mesh=mesh,
        out_type=jax.ShapeDtypeStruct((TOK, HID), jnp.float32),
        compiler_params=pltpu.CompilerParams(needs_layout_passes=False),
        scratch_types=[
            pltpu.VMEM((NCHUNK, CHUNK), jnp.int32),
            pltpu.VMEM((NCHUNK, CHUNK), jnp.int32),
            pltpu.VMEM((TOK_PER_W,), jnp.float32),
            pltpu.VMEM((2, HID), jnp.float32),
            pltpu.VMEM((CHUNK, HID), jnp.float32),
            pltpu.VMEM((CHUNK, HID), jnp.float32),
            pltpu.VMEM((CHUNK, HID), jnp.float32),
            pltpu.VMEM((CHUNK, HID), jnp.float32),
            pltpu.SemaphoreType.DMA,
            pltpu.SemaphoreType.DMA,
            pltpu.SemaphoreType.DMA,
            pltpu.SemaphoreType.DMA,
            pltpu.SemaphoreType.DMA,
            pltpu.SemaphoreType.DMA,
        ],
    )(_sc_body)

    out = run(ids, pids, tidsf,
              word_embeddings.astype(jnp.float32),
              position_embeddings.astype(jnp.float32),
              token_type_embeddings.astype(jnp.float32))
    return out.reshape(b, s, HID)


# speedup vs baseline: 4.5999x; 1.0986x over previous
import functools

import jax
import jax.numpy as jnp
from jax import lax
from jax.experimental import pallas as pl
from jax.experimental.pallas import tpu as pltpu
from jax.experimental.pallas import tpu_sc as plsc

HID = 768
TOK = 4 * 2048

NC = 2
NS = 16
NW = NC * NS
TOK_PER_W = TOK // NW
CHUNK = 32
NCHUNK = TOK_PER_W // CHUNK
LANES = 16
VPT = HID // LANES
NBW = 3


def _sc_body(ids_hbm, pids_hbm, tids_hbm, wtab, ptab, ttab, out_hbm,
             idx_w, idx_p, tf_vmem, type_v,
             bufw0, bufw1, bufw2, bufp0, bufp1,
             semw0, semw1, semw2, semp0, semp1,
             semo0, semo1, semo2, semst):
    wid = lax.axis_index("s") * NC + lax.axis_index("c")
    row0 = wid * NCHUNK
    base = wid * TOK_PER_W

    bufw = (bufw0, bufw1, bufw2)
    bufp = (bufp0, bufp1)
    semw = (semw0, semw1, semw2)
    semp = (semp0, semp1)
    semo = (semo0, semo1, semo2)

    st1 = pltpu.async_copy(ids_hbm.at[pl.ds(row0, NCHUNK)], idx_w, semst)
    st2 = pltpu.async_copy(pids_hbm.at[pl.ds(row0, NCHUNK)], idx_p, semst)
    st3 = pltpu.async_copy(tids_hbm.at[pl.ds(base, TOK_PER_W)], tf_vmem, semst)
    st4 = pltpu.async_copy(ttab, type_v, semst)

    def start_gathers(c):
        cw = pltpu.async_copy(wtab.at[idx_w.at[c]], bufw[c % NBW],
                              semw[c % NBW])
        cp = pltpu.async_copy(ptab.at[idx_p.at[c]], bufp[c % 2], semp[c % 2])
        return cw, cp

    def compute(c):
        bw, bp = c % NBW, c % 2

        def add_k(k, carry):
            sl = pl.ds(k * LANES, LANES)
            t0k = type_v[0, sl]
            dk = type_v[1, sl] - t0k
            for g in range(CHUNK // LANES):
                tfv = tf_vmem[pl.ds(c * CHUNK + g * LANES, LANES)].astype(
                    jnp.float32)
                for l in range(LANES):
                    j = g * LANES + l
                    tf = jnp.full((LANES,), tfv[l], jnp.float32)
                    bufw[bw][j, sl] = (bufw[bw][j, sl] + bufp[bp][j, sl]
                                       + (t0k + tf * dk))
            return carry

        lax.fori_loop(0, VPT, add_k, 0)

    st1.wait()
    st2.wait()
    gath = {0: start_gathers(0), 1: start_gathers(1)}
    st3.wait()
    st4.wait()
    O = {}
    for c in range(NCHUNK):
        cw, cp = gath.pop(c)
        cw.wait()
        cp.wait()
        compute(c)
        O[c] = pltpu.async_copy(
            bufw[c % NBW], out_hbm.at[pl.ds(base + c * CHUNK, CHUNK)],
            semo[c % NBW])
        if c + 2 < NCHUNK:
            if c - 1 >= 0:
                O[c - 1].wait()
            gath[c + 2] = start_gathers(c + 2)
    for c in range(NCHUNK - 3, NCHUNK):
        O[c].wait()


def kernel(input_ids, position_ids, token_type_ids, word_embeddings,
           token_type_embeddings, position_embeddings):
    b, s = input_ids.shape
    ids = input_ids.reshape(TOK // CHUNK, CHUNK).astype(jnp.int32)
    pids = position_ids.reshape(TOK // CHUNK, CHUNK).astype(jnp.int32)
    tids = token_type_ids.reshape(TOK).astype(jnp.int32)

    mesh = plsc.VectorSubcoreMesh(core_axis_name="c", subcore_axis_name="s")
    run = functools.partial(
        pl.kernel,
        mesh=mesh,
        out_type=jax.ShapeDtypeStruct((TOK, HID), jnp.float32),
        compiler_params=pltpu.CompilerParams(needs_layout_passes=False),
        scratch_types=[
            pltpu.VMEM((NCHUNK, CHUNK), jnp.int32),
            pltpu.VMEM((NCHUNK, CHUNK), jnp.int32),
            pltpu.VMEM((TOK_PER_W,), jnp.int32),
            pltpu.VMEM((2, HID), jnp.float32),
            pltpu.VMEM((CHUNK, HID), jnp.float32),
            pltpu.VMEM((CHUNK, HID), jnp.float32),
            pltpu.VMEM((CHUNK, HID), jnp.float32),
            pltpu.VMEM((CHUNK, HID), jnp.float32),
            pltpu.VMEM((CHUNK, HID), jnp.float32),
            pltpu.SemaphoreType.DMA,
            pltpu.SemaphoreType.DMA,
            pltpu.SemaphoreType.DMA,
            pltpu.SemaphoreType.DMA,
            pltpu.SemaphoreType.DMA,
            pltpu.SemaphoreType.DMA,
            pltpu.SemaphoreType.DMA,
            pltpu.SemaphoreType.DMA,
            pltpu.SemaphoreType.DMA,
        ],
    )(_sc_body)

    out = run(ids, pids, tids,
              word_embeddings.astype(jnp.float32),
              position_embeddings.astype(jnp.float32),
              token_type_embeddings.astype(jnp.float32))
    return out.reshape(b, s, HID)
